# baseline (speedup 1.0000x reference)
import jax
import jax.numpy as jnp
from jax import lax
from jax.experimental import pallas as pl
from jax.experimental.pallas import tpu as pltpu
from jax.experimental.pallas import tpu_sc as plsc

_NC = 2
_NS = 16
_L = 16
_RPC = 16


def kernel(token_ids, table):
    B, H = token_ids.shape
    V, D = table.shape
    idx_flat = token_ids.reshape(-1)
    tab_flat = table.reshape(-1)

    nw = _NC * _NS
    b_per_w = B // nw
    chunks = b_per_w // _RPC
    chunk_ids = _RPC * H

    offs = list(range(0, 2 * H, _L))

    mesh = plsc.VectorSubcoreMesh(core_axis_name="core", subcore_axis_name="subcore")

    @pl.kernel(
        out_type=jax.ShapeDtypeStruct((B, H, D), jnp.float32),
        mesh=mesh,
        scratch_types=[
            pltpu.VMEM((V * D,), jnp.float32),
            pltpu.VMEM((2, chunk_ids), jnp.int32),
            pltpu.VMEM((2, H, D), jnp.float32),
            pltpu.SemaphoreType.DMA,
        ],
        compiler_params=pltpu.CompilerParams(use_tc_tiling_on_sc=True),
    )
    def emb_kernel(tab_hbm, idx_hbm, out_hbm, tab_v, idx_v, o_v, sem):
        wid = lax.axis_index("subcore") * _NC + lax.axis_index("core")
        b0 = wid * b_per_w
        pltpu.async_copy(tab_hbm, tab_v, sem).wait()

        @pl.loop(0, chunks)
        def _(g):
            pltpu.sync_copy(
                idx_hbm.at[pl.ds((b0 + g * _RPC) * H, chunk_ids)], idx_v.at[0]
            )

            @pl.loop(0, _RPC // 2)
            def _(r2):
                for o in offs:
                    tv = idx_v[0, pl.ds(r2 * (2 * H) + o, _L)] * D
                    for j in range(_L):
                        off = tv[j]
                        t = o + j
                        o_v[t // H, t % H, pl.ds(0, _L)] = tab_v[pl.ds(off, _L)]
                        o_v[t // H, t % H, pl.ds(_L, _L)] = tab_v[pl.ds(off + _L, _L)]
                pltpu.sync_copy(
                    o_v, out_hbm.at[pl.ds(b0 + g * _RPC + r2 * 2, 2)]
                )

    out = emb_kernel(tab_flat, idx_flat)
    return out


# speedup vs baseline: 1.2399x; 1.2399x over previous
import jax
import jax.numpy as jnp
from jax import lax
from jax.experimental import pallas as pl
from jax.experimental.pallas import tpu as pltpu
from jax.experimental.pallas import tpu_sc as plsc

_NC = 2
_NS = 16
_L = 16
_HB = 8
_BB = 128


def kernel(token_ids, table):
    B, H = token_ids.shape
    V, D = table.shape
    idx_t = token_ids.T
    tab_flat = table.reshape(-1)

    nw = _NC * _NS
    bblocks_per_w = B // _BB // nw
    hblocks = H // _HB
    nblocks = bblocks_per_w * hblocks

    mesh = plsc.VectorSubcoreMesh(core_axis_name="core", subcore_axis_name="subcore")

    @pl.kernel(
        out_type=jax.ShapeDtypeStruct((H, D, B), jnp.float32),
        mesh=mesh,
        scratch_types=[
            pltpu.VMEM((V * D,), jnp.float32),
            pltpu.VMEM((2, _HB, _BB), jnp.int32),
            pltpu.VMEM((2, _HB, D, _BB), jnp.float32),
            pltpu.SemaphoreType.DMA,
            pltpu.SemaphoreType.DMA,
            pltpu.SemaphoreType.DMA,
        ],
        compiler_params=pltpu.CompilerParams(
            use_tc_tiling_on_sc=True, needs_layout_passes=False
        ),
    )
    def emb_kernel(tab_hbm, idx_hbm, out_hbm, tab_v, idx_v, o_v, sem_t, sem_i, sem_o):
        wid = lax.axis_index("subcore") * _NC + lax.axis_index("core")
        b0 = wid * (bblocks_per_w * _BB)
        pltpu.async_copy(tab_hbm, tab_v, sem_t).wait()

        c_lo = lax.iota(jnp.int32, _L)
        c_hi = c_lo + _L

        def blk_slices(blk):
            hb = lax.rem(blk, hblocks)
            bb = blk // hblocks
            return pl.ds(hb * _HB, _HB), pl.ds(b0 + bb * _BB, _BB)

        def idx_dma(blk, buf):
            hs, bs = blk_slices(blk)
            return pltpu.make_async_copy(idx_hbm.at[hs, bs], idx_v.at[buf], sem_i)

        def out_dma(blk, buf):
            hs, bs = blk_slices(blk)
            return pltpu.make_async_copy(
                o_v.at[buf], out_hbm.at[hs, :, bs], sem_o
            )

        idx_dma(0, 0).start()

        @pl.loop(0, nblocks)
        def _(blk):
            buf = lax.rem(blk, 2)
            nxt = 1 - buf

            @pl.when(blk + 1 < nblocks)
            def _():
                idx_dma(blk + 1, nxt).start()

            idx_dma(blk, buf).wait()

            @pl.when(blk >= 2)
            def _():
                out_dma(blk - 2, buf).wait()

            @pl.loop(0, _HB * (_BB // _L))
            def _(q):
                h = q // (_BB // _L)
                bq = lax.rem(q, _BB // _L)
                tv = idx_v[buf, h, pl.ds(bq * _L, _L)] * D
                h_splat = jnp.full((_L,), h, jnp.int32)
                for j in range(_L):
                    off = tv[j]
                    b_splat = jnp.full((_L,), bq * _L + j, jnp.int32)
                    plsc.store_scatter(
                        o_v.at[buf],
                        [h_splat, c_lo, b_splat],
                        tab_v[pl.ds(off, _L)],
                    )
                    plsc.store_scatter(
                        o_v.at[buf],
                        [h_splat, c_hi, b_splat],
                        tab_v[pl.ds(off + _L, _L)],
                    )

            out_dma(blk, buf).start()

        out_dma(nblocks - 2, 0).wait()
        out_dma(nblocks - 1, 1).wait()

    out = emb_kernel(tab_flat, idx_t)
    return jnp.transpose(out, (2, 0, 1))
